# baseline (speedup 1.0000x reference)
import functools

import jax
import jax.numpy as jnp
from jax import lax
from jax.experimental import pallas as pl
from jax.experimental.pallas import tpu as pltpu
from jax.experimental.pallas import tpu_sc as plsc

_R = 16384
_C = 4096
_NC = 2
_NS = 16
_NW = _NC * _NS
_ROWS_PER_W = _R // _NW
_CHUNK_ROWS = 4
_CHUNK = _CHUNK_ROWS * _C
_NCHUNKS = _ROWS_PER_W // _CHUNK_ROWS
_NBLK = _CHUNK // 64
_NBUF = 2
_NG = _NCHUNKS // _NBUF


def _prune_chunk(in_v, out_v):
    iota = lax.iota(jnp.int32, 16)
    pos = [(iota + k) % 4 for k in range(4)]
    bases = [iota * 4 + pos[k] for k in range(4)]
    m01 = pos[0] < pos[1]
    m02 = pos[0] < pos[2]
    m03 = pos[0] < pos[3]
    m12 = pos[1] < pos[2]
    m13 = pos[1] < pos[3]
    m23 = pos[2] < pos[3]

    @plsc.parallel_loop(0, _NBLK, unroll=4)
    def blk(i):
        o = i * 64
        idx0 = bases[0] + o
        idx1 = bases[1] + o
        idx2 = bases[2] + o
        idx3 = bases[3] + o
        e0 = plsc.load_gather(in_v, [idx0])
        e1 = plsc.load_gather(in_v, [idx1])
        e2 = plsc.load_gather(in_v, [idx2])
        e3 = plsc.load_gather(in_v, [idx3])
        a0 = jnp.abs(e0)
        a1 = jnp.abs(e1)
        a2 = jnp.abs(e2)
        a3 = jnp.abs(e3)
        b01 = ((a0 > a1) | ((a0 == a1) & m01)).astype(jnp.int32)
        b02 = ((a0 > a2) | ((a0 == a2) & m02)).astype(jnp.int32)
        b03 = ((a0 > a3) | ((a0 == a3) & m03)).astype(jnp.int32)
        b12 = ((a1 > a2) | ((a1 == a2) & m12)).astype(jnp.int32)
        b13 = ((a1 > a3) | ((a1 == a3) & m13)).astype(jnp.int32)
        b23 = ((a2 > a3) | ((a2 == a3) & m23)).astype(jnp.int32)
        c0 = 3 - b01 - b02 - b03
        c1 = (2 + b01) - b12 - b13
        c2 = (b02 + b12) + (1 - b23)
        c3 = b03 + b13 + b23
        zero = jnp.zeros((16,), jnp.float32)
        plsc.store_scatter(out_v, [idx0], jnp.where(c0 < 2, e0, zero))
        plsc.store_scatter(out_v, [idx1], jnp.where(c1 < 2, e1, zero))
        plsc.store_scatter(out_v, [idx2], jnp.where(c2 < 2, e2, zero))
        plsc.store_scatter(out_v, [idx3], jnp.where(c3 < 2, e3, zero))


@functools.partial(
    pl.kernel,
    mesh=plsc.VectorSubcoreMesh(core_axis_name="c", subcore_axis_name="s"),
    out_type=jax.ShapeDtypeStruct((_R * _C,), jnp.float32),
    scratch_types=[
        [pltpu.VMEM((_CHUNK,), jnp.float32) for _ in range(_NBUF)],
        [pltpu.VMEM((_CHUNK,), jnp.float32) for _ in range(_NBUF)],
        [pltpu.SemaphoreType.DMA for _ in range(_NBUF)],
        [pltpu.SemaphoreType.DMA for _ in range(_NBUF)],
    ],
    compiler_params=pltpu.CompilerParams(needs_layout_passes=False),
)
def _prune_sc(w_hbm, out_hbm, in_vs, out_vs, in_sems, out_sems):
    wid = lax.axis_index("s") * _NC + lax.axis_index("c")
    base = wid * (_ROWS_PER_W * _C)

    for b in range(_NBUF):
        pltpu.async_copy(
            w_hbm.at[pl.ds(base + b * _CHUNK, _CHUNK)], in_vs[b], in_sems[b]
        )

    def g_body(g, _):
        for b in range(_NBUF):
            ci = g * _NBUF + b
            off = base + ci * _CHUNK
            pltpu.make_async_copy(
                w_hbm.at[pl.ds(off, _CHUNK)], in_vs[b], in_sems[b]
            ).wait()
            @pl.when(g > 0)
            def _wait_out():
                pltpu.make_async_copy(
                    out_vs[b],
                    out_hbm.at[pl.ds(off - _NBUF * _CHUNK, _CHUNK)],
                    out_sems[b],
                ).wait()

            _prune_chunk(in_vs[b], out_vs[b])
            pltpu.async_copy(
                out_vs[b], out_hbm.at[pl.ds(off, _CHUNK)], out_sems[b]
            )

            @pl.when(g < _NG - 1)
            def _next_in():
                pltpu.async_copy(
                    w_hbm.at[pl.ds(off + _NBUF * _CHUNK, _CHUNK)],
                    in_vs[b],
                    in_sems[b],
                )

        return 0

    lax.fori_loop(0, _NG, g_body, 0)

    for b in range(_NBUF):
        off = base + ((_NG - 1) * _NBUF + b) * _CHUNK
        pltpu.make_async_copy(
            out_vs[b], out_hbm.at[pl.ds(off, _CHUNK)], out_sems[b]
        ).wait()


@jax.jit
def kernel(weight):
    out = _prune_sc(weight.reshape(_R * _C))
    return out.reshape(_R, _C)


# speedup vs baseline: 1.5007x; 1.5007x over previous
import functools

import jax
import jax.numpy as jnp
from jax import lax
from jax.experimental import pallas as pl
from jax.experimental.pallas import tpu as pltpu
from jax.experimental.pallas import tpu_sc as plsc

_R = 16384
_C = 4096
_NC = 2
_NS = 16
_NW = _NC * _NS
_ROWS_PER_W = _R // _NW
_CHUNK_ROWS = 4
_CHUNK = _CHUNK_ROWS * _C
_NCHUNKS = _ROWS_PER_W // _CHUNK_ROWS
_NBLK = _CHUNK // 64
_NBUF = 2
_NG = _NCHUNKS // _NBUF


def _prune_chunk(in_v, out_v):
    iota = lax.iota(jnp.int32, 16)
    base0 = iota * 4
    base1 = base0 + 1
    base2 = base0 + 2
    base3 = base0 + 3

    @plsc.parallel_loop(0, _CHUNK, step=64, unroll=4)
    def blk(o):
        in_b = in_v.at[pl.ds(o, 64)]
        out_b = out_v.at[pl.ds(o, 64)]
        e0 = plsc.load_gather(in_b, [base0])
        e1 = plsc.load_gather(in_b, [base1])
        e2 = plsc.load_gather(in_b, [base2])
        e3 = plsc.load_gather(in_b, [base3])
        a0 = jnp.abs(e0)
        a1 = jnp.abs(e1)
        a2 = jnp.abs(e2)
        a3 = jnp.abs(e3)
        g10 = (a1 > a0).astype(jnp.int32)
        g20 = (a2 > a0).astype(jnp.int32)
        g30 = (a3 > a0).astype(jnp.int32)
        g21 = (a2 > a1).astype(jnp.int32)
        g31 = (a3 > a1).astype(jnp.int32)
        g32 = (a3 > a2).astype(jnp.int32)
        c0 = g10 + g20 + g30
        c1 = (1 - g10) + g21 + g31
        c2 = (2 - g20 - g21) + g32
        c3 = 3 - g30 - g31 - g32
        zero = jnp.zeros((16,), jnp.float32)
        plsc.store_scatter(out_b, [base0], jnp.where(c0 < 2, e0, zero))
        plsc.store_scatter(out_b, [base1], jnp.where(c1 < 2, e1, zero))
        plsc.store_scatter(out_b, [base2], jnp.where(c2 < 2, e2, zero))
        plsc.store_scatter(out_b, [base3], jnp.where(c3 < 2, e3, zero))


@functools.partial(
    pl.kernel,
    mesh=plsc.VectorSubcoreMesh(core_axis_name="c", subcore_axis_name="s"),
    out_type=jax.ShapeDtypeStruct((_R * _C,), jnp.float32),
    scratch_types=[
        [pltpu.VMEM((_CHUNK,), jnp.float32) for _ in range(_NBUF)],
        [pltpu.VMEM((_CHUNK,), jnp.float32) for _ in range(_NBUF)],
        [pltpu.SemaphoreType.DMA for _ in range(_NBUF)],
        [pltpu.SemaphoreType.DMA for _ in range(_NBUF)],
    ],
    compiler_params=pltpu.CompilerParams(needs_layout_passes=False),
)
def _prune_sc(w_hbm, out_hbm, in_vs, out_vs, in_sems, out_sems):
    wid = lax.axis_index("s") * _NC + lax.axis_index("c")
    base = wid * (_ROWS_PER_W * _C)

    for b in range(_NBUF):
        pltpu.async_copy(
            w_hbm.at[pl.ds(base + b * _CHUNK, _CHUNK)], in_vs[b], in_sems[b]
        )

    def g_body(g, _):
        for b in range(_NBUF):
            ci = g * _NBUF + b
            off = base + ci * _CHUNK
            pltpu.make_async_copy(
                w_hbm.at[pl.ds(off, _CHUNK)], in_vs[b], in_sems[b]
            ).wait()
            @pl.when(g > 0)
            def _wait_out():
                pltpu.make_async_copy(
                    out_vs[b],
                    out_hbm.at[pl.ds(off - _NBUF * _CHUNK, _CHUNK)],
                    out_sems[b],
                ).wait()

            _prune_chunk(in_vs[b], out_vs[b])
            pltpu.async_copy(
                out_vs[b], out_hbm.at[pl.ds(off, _CHUNK)], out_sems[b]
            )

            @pl.when(g < _NG - 1)
            def _next_in():
                pltpu.async_copy(
                    w_hbm.at[pl.ds(off + _NBUF * _CHUNK, _CHUNK)],
                    in_vs[b],
                    in_sems[b],
                )

        return 0

    lax.fori_loop(0, _NG, g_body, 0)

    for b in range(_NBUF):
        off = base + ((_NG - 1) * _NBUF + b) * _CHUNK
        pltpu.make_async_copy(
            out_vs[b], out_hbm.at[pl.ds(off, _CHUNK)], out_sems[b]
        ).wait()


@jax.jit
def kernel(weight):
    out = _prune_sc(weight.reshape(_R * _C))
    return out.reshape(_R, _C)


# speedup vs baseline: 1.8190x; 1.2121x over previous
import functools

import jax
import jax.numpy as jnp
from jax import lax
from jax.experimental import pallas as pl
from jax.experimental.pallas import tpu as pltpu
from jax.experimental.pallas import tpu_sc as plsc

_R = 16384
_C = 4096
_NC = 2
_NS = 16
_NW = _NC * _NS
_ROWS_PER_W = _R // _NW
_CHUNK_ROWS = 4
_CHUNK = _CHUNK_ROWS * _C
_NCHUNKS = _ROWS_PER_W // _CHUNK_ROWS
_NBLK = _CHUNK // 64
_NBUF = 2
_NG = _NCHUNKS // _NBUF


def _prune_chunk(in_v, out_v):
    iota = lax.iota(jnp.int32, 16)
    base0 = iota * 4
    base1 = base0 + 1
    base2 = base0 + 2
    base3 = base0 + 3

    @plsc.parallel_loop(0, _CHUNK, step=64, unroll=4)
    def blk(o):
        in_b = in_v.at[pl.ds(o, 64)]
        out_b = out_v.at[pl.ds(o, 64)]
        e0 = plsc.load_gather(in_b, [base0])
        e1 = plsc.load_gather(in_b, [base1])
        e2 = plsc.load_gather(in_b, [base2])
        e3 = plsc.load_gather(in_b, [base3])
        a0 = jnp.abs(e0)
        a1 = jnp.abs(e1)
        a2 = jnp.abs(e2)
        a3 = jnp.abs(e3)
        g10 = (a1 > a0).astype(jnp.int32)
        g20 = (a2 > a0).astype(jnp.int32)
        g30 = (a3 > a0).astype(jnp.int32)
        g21 = (a2 > a1).astype(jnp.int32)
        g31 = (a3 > a1).astype(jnp.int32)
        g32 = (a3 > a2).astype(jnp.int32)
        c0 = g10 + g20 + g30
        c1 = (1 - g10) + g21 + g31
        c2 = (2 - g20 - g21) + g32
        c3 = 3 - g30 - g31 - g32
        zero = jnp.zeros((16,), jnp.float32)
        plsc.store_scatter(out_b, [base0], jnp.where(c0 < 2, e0, zero))
        plsc.store_scatter(out_b, [base1], jnp.where(c1 < 2, e1, zero))
        plsc.store_scatter(out_b, [base2], jnp.where(c2 < 2, e2, zero))
        plsc.store_scatter(out_b, [base3], jnp.where(c3 < 2, e3, zero))


@functools.partial(
    pl.kernel,
    mesh=plsc.VectorSubcoreMesh(core_axis_name="c", subcore_axis_name="s"),
    out_type=jax.ShapeDtypeStruct((_R * _C,), jnp.float32),
    scratch_types=[
        [pltpu.VMEM((_CHUNK,), jnp.float32) for _ in range(_NBUF)],
        [pltpu.VMEM((_CHUNK,), jnp.float32) for _ in range(_NBUF)],
        [pltpu.SemaphoreType.DMA for _ in range(_NBUF)],
        [pltpu.SemaphoreType.DMA for _ in range(_NBUF)],
    ],
    compiler_params=pltpu.CompilerParams(needs_layout_passes=False),
)
def _prune_sc(w_hbm, out_hbm, in_vs, out_vs, in_sems, out_sems):
    wid = lax.axis_index("s") * _NC + lax.axis_index("c")
    base = wid * (_ROWS_PER_W * _C)

    for b in range(_NBUF):
        pltpu.async_copy(
            w_hbm.at[pl.ds(base + b * _CHUNK, _CHUNK)], in_vs[b], in_sems[b]
        )

    def g_body(g, _):
        for b in range(_NBUF):
            ci = g * _NBUF + b
            off = base + ci * _CHUNK
            pltpu.make_async_copy(
                w_hbm.at[pl.ds(off, _CHUNK)], in_vs[b], in_sems[b]
            ).wait()
            @pl.when(g > 0)
            def _wait_out():
                pltpu.make_async_copy(
                    out_vs[b],
                    out_hbm.at[pl.ds(off - _NBUF * _CHUNK, _CHUNK)],
                    out_sems[b],
                ).wait()

            _prune_chunk(in_vs[b], out_vs[b])
            pltpu.async_copy(
                out_vs[b], out_hbm.at[pl.ds(off, _CHUNK)], out_sems[b]
            )

            @pl.when(g < _NG - 1)
            def _next_in():
                pltpu.async_copy(
                    w_hbm.at[pl.ds(off + _NBUF * _CHUNK, _CHUNK)],
                    in_vs[b],
                    in_sems[b],
                )

        return 0

    lax.fori_loop(0, _NG, g_body, 0)

    for b in range(_NBUF):
        off = base + ((_NG - 1) * _NBUF + b) * _CHUNK
        pltpu.make_async_copy(
            out_vs[b], out_hbm.at[pl.ds(off, _CHUNK)], out_sems[b]
        ).wait()


_BR = 256


def _tc_body(x_ref, o_ref):
    x = x_ref[...]
    a = jnp.abs(x)
    p = jax.lax.broadcasted_iota(jnp.int32, x.shape, 1) % 4
    cnt = jnp.zeros(x.shape, jnp.int32)
    for s in (1, 2, 3):
        fwd = pltpu.roll(a, x.shape[1] - s, 1)
        bwd = pltpu.roll(a, s, 1)
        vf = p + s <= 3
        vb = p - s >= 0
        cnt = cnt + jnp.where(vf & (fwd > a), 1, 0)
        cnt = cnt + jnp.where(vb & (bwd >= a), 1, 0)
    o_ref[...] = jnp.where(cnt < 2, x, 0.0)


def _tc_prune(w):
    nrows = w.shape[0]
    return pl.pallas_call(
        _tc_body,
        grid=(nrows // _BR,),
        in_specs=[pl.BlockSpec((_BR, _C), lambda i: (i, 0))],
        out_specs=pl.BlockSpec((_BR, _C), lambda i: (i, 0)),
        out_shape=jax.ShapeDtypeStruct((nrows, _C), jnp.float32),
    )(w)


@jax.jit
def kernel(weight):
    return _tc_prune(weight)
